# baseline (speedup 1.0000x reference)
import jax
import jax.numpy as jnp
from jax.experimental import pallas as pl
from jax.experimental.pallas import tpu as pltpu

_B, _H, _W, _J = 32, 512, 512, 17


def _body(xs_ref, ys_ref, m_ref, g_ref, out_ref):
    b = pl.program_id(0)

    @pl.when(b == 0)
    def _init():
        for i in range(4):
            out_ref[i] = 0.0

    xrow = xs_ref[pl.ds(b, 1), :]
    yrow = ys_ref[pl.ds(b, 1), :]
    x_min = jnp.maximum(jnp.min(xrow).astype(jnp.int32) - 10, 0)
    x_max = jnp.minimum(jnp.max(xrow).astype(jnp.int32) + 10, _W)
    y_min = jnp.maximum(jnp.min(yrow).astype(jnp.int32) - 10, 0)
    y_max = jnp.minimum(jnp.max(yrow).astype(jnp.int32) + 10, _H)

    x = m_ref[...]
    g = g_ref[...]

    l_term = jnp.log1p(jnp.exp(-jnp.abs(x)))
    pos_val = jnp.maximum(-x, 0.0) + l_term
    neg_val = jnp.maximum(x, 0.0) + l_term

    posf = (g > 0.0).astype(jnp.float32)
    rows = jax.lax.broadcasted_iota(jnp.int32, x.shape, 1)
    cols = jax.lax.broadcasted_iota(jnp.int32, x.shape, 2)
    inside = (rows >= y_min) & (rows < y_max) & (cols >= x_min) & (cols < x_max)
    negf = 1.0 - inside.astype(jnp.float32)

    out_ref[0] += jnp.sum(pos_val * posf)
    out_ref[1] += jnp.sum(posf)
    out_ref[2] += jnp.sum(neg_val * negf)
    out_ref[3] += jnp.sum(negf)


def kernel(skls, masks, gt_masks):
    s = jax.lax.stop_gradient(skls)
    xs = s[:, :, 0]
    ys = s[:, :, 1]

    acc = pl.pallas_call(
        _body,
        grid=(_B,),
        in_specs=[
            pl.BlockSpec((_B, _J), lambda b: (0, 0)),
            pl.BlockSpec((_B, _J), lambda b: (0, 0)),
            pl.BlockSpec((1, _H, _W), lambda b: (b, 0, 0)),
            pl.BlockSpec((1, _H, _W), lambda b: (b, 0, 0)),
        ],
        out_specs=pl.BlockSpec(memory_space=pltpu.SMEM),
        out_shape=jax.ShapeDtypeStruct((4,), jnp.float32),
        compiler_params=pltpu.CompilerParams(
            dimension_semantics=("arbitrary",),
        ),
    )(xs, ys, masks, gt_masks)

    pos_loss = acc[0] / jnp.maximum(acc[1], 1.0)
    neg_loss = acc[2] / jnp.maximum(acc[3], 1.0)
    return pos_loss + neg_loss


# speedup vs baseline: 1.0379x; 1.0379x over previous
import jax
import jax.numpy as jnp
from jax.experimental import pallas as pl
from jax.experimental.pallas import tpu as pltpu

_B, _H, _W, _J = 32, 512, 512, 17


def _body(xs_ref, ys_ref, m_ref, g_ref, out_ref):
    b = pl.program_id(0)

    @pl.when(b == 0)
    def _init():
        for i in range(4):
            out_ref[i] = 0.0

    xrow = xs_ref[pl.ds(b, 1), :]
    yrow = ys_ref[pl.ds(b, 1), :]
    x_min = jnp.maximum(jnp.min(xrow).astype(jnp.int32) - 10, 0)
    x_max = jnp.minimum(jnp.max(xrow).astype(jnp.int32) + 10, _W)
    y_min = jnp.maximum(jnp.min(yrow).astype(jnp.int32) - 10, 0)
    y_max = jnp.minimum(jnp.max(yrow).astype(jnp.int32) + 10, _H)

    x = m_ref[...]
    g = g_ref[...]

    l_term = jnp.log1p(jnp.exp(-jnp.abs(x)))
    pos_val = jnp.maximum(-x, 0.0) + l_term
    neg_val = pos_val + x

    zero = jnp.zeros_like(x)
    pos = g > 0.0

    y_len = jnp.maximum(y_max - y_min, 0).astype(jnp.uint32)
    x_len = jnp.maximum(x_max - x_min, 0).astype(jnp.uint32)
    rows = jax.lax.broadcasted_iota(jnp.int32, x.shape, 1)
    cols = jax.lax.broadcasted_iota(jnp.int32, x.shape, 2)
    inside = ((rows - y_min).astype(jnp.uint32) < y_len) & (
        (cols - x_min).astype(jnp.uint32) < x_len)

    out_ref[0] += jnp.sum(jnp.where(pos, pos_val, zero))
    out_ref[1] += jnp.sum(jnp.where(pos, 1.0, 0.0))
    out_ref[2] += jnp.sum(jnp.where(inside, zero, neg_val))
    out_ref[3] += (y_len * x_len).astype(jnp.float32)


def kernel(skls, masks, gt_masks):
    s = jax.lax.stop_gradient(skls)
    xs = s[:, :, 0]
    ys = s[:, :, 1]

    acc = pl.pallas_call(
        _body,
        grid=(_B,),
        in_specs=[
            pl.BlockSpec((_B, _J), lambda b: (0, 0)),
            pl.BlockSpec((_B, _J), lambda b: (0, 0)),
            pl.BlockSpec((1, _H, _W), lambda b: (b, 0, 0)),
            pl.BlockSpec((1, _H, _W), lambda b: (b, 0, 0)),
        ],
        out_specs=pl.BlockSpec(memory_space=pltpu.SMEM),
        out_shape=jax.ShapeDtypeStruct((4,), jnp.float32),
        compiler_params=pltpu.CompilerParams(
            dimension_semantics=("arbitrary",),
        ),
    )(xs, ys, masks, gt_masks)

    pos_loss = acc[0] / jnp.maximum(acc[1], 1.0)
    neg_count = float(_B * _H * _W) - acc[3]
    neg_loss = acc[2] / jnp.maximum(neg_count, 1.0)
    return pos_loss + neg_loss
